# Initial kernel scaffold; baseline (speedup 1.0000x reference)
import jax
import jax.numpy as jnp
from jax.experimental import pallas as pl


def kernel(features, edge_index, Ws0, Wn0, b0, Ws1, Wn1, b1, Ws2, Wn2, b2):
    raise NotImplementedError("write your pallas kernel here")


# speedup vs baseline: 8.0758x; 8.0758x over previous
import functools

import jax
import jax.numpy as jnp
from jax import lax
from jax.experimental import pallas as pl
from jax.experimental.pallas import tpu as pltpu
from jax.experimental.pallas import tpu_sc as plsc

N = 10000
E = 320000
D = 128
NPAD = 10240
NC = 2
NS = 16
NW = NC * NS
EPT = E // NW
CH = 128
NCHUNK = -(-EPT // CH)
EPTP = NCHUNK * CH
RPT = NPAD // NS

_MESH = plsc.VectorSubcoreMesh(
    core_axis_name="c", subcore_axis_name="s", num_cores=NC, num_subcores=NS)


def _fill(buf, rows, val):
  v = jnp.full((16,), val, jnp.float32)

  def body(i, _):
    for k in range(D // 16):
      buf[i, pl.ds(k * 16, 16)] = v
    return 0

  lax.fori_loop(0, rows, body, 0)


def _zero_acc(acc, rows_buf, sid):
  _fill(rows_buf, CH, 0.0)
  for k in range(RPT // CH):
    pltpu.sync_copy(rows_buf, acc.at[pl.ds(sid * RPT + k * CH, CH), :])


def _writeout(acc, out_hbm, cid, sid):
  pltpu.sync_copy(acc.at[pl.ds(sid * RPT, RPT), :],
                  out_hbm.at[cid, pl.ds(sid * RPT, RPT), :])


@functools.partial(
    pl.kernel,
    out_type=jax.ShapeDtypeStruct((NC, NPAD, D), jnp.float32),
    mesh=_MESH,
    scratch_types=[
        pltpu.VMEM((NCHUNK, CH), jnp.int32),
        pltpu.VMEM((CH, D), jnp.float32),
        pltpu.MemorySpace.VMEM_SHARED((NPAD, D), jnp.float32),
    ],
)
def _deg_kernel(dst3_hbm, out_hbm, dstbuf, rows_buf, acc):
  cid = lax.axis_index("c")
  sid = lax.axis_index("s")
  wid = cid * NS + sid
  pltpu.sync_copy(dst3_hbm.at[wid], dstbuf)
  _zero_acc(acc, rows_buf, sid)
  plsc.subcore_barrier()
  _fill(rows_buf, CH, 1.0)

  def chunk(j, _):
    pltpu.sync_copy(rows_buf, acc.at[dstbuf.at[j]], add=True)
    return 0

  lax.fori_loop(0, NCHUNK, chunk, 0)
  plsc.subcore_barrier()
  _writeout(acc, out_hbm, cid, sid)


@functools.partial(
    pl.kernel,
    out_type=jax.ShapeDtypeStruct((NC, NPAD, D), jnp.float32),
    mesh=_MESH,
    scratch_types=[
        pltpu.VMEM((NCHUNK, CH), jnp.int32),
        pltpu.VMEM((NCHUNK, CH), jnp.int32),
        pltpu.VMEM((CH, D), jnp.float32),
        pltpu.MemorySpace.VMEM_SHARED((NPAD, D), jnp.float32),
        pltpu.SemaphoreType.DMA,
    ],
)
def _agg_kernel(h_hbm, src3_hbm, dst3_hbm, out_hbm, srcbuf, dstbuf, rows_buf,
                acc, sem):
  cid = lax.axis_index("c")
  sid = lax.axis_index("s")
  wid = cid * NS + sid
  pltpu.sync_copy(src3_hbm.at[wid], srcbuf)
  pltpu.sync_copy(dst3_hbm.at[wid], dstbuf)
  _zero_acc(acc, rows_buf, sid)
  plsc.subcore_barrier()

  def chunk(j, _):
    pltpu.async_copy(h_hbm.at[srcbuf.at[j]], rows_buf, sem).wait()
    pltpu.sync_copy(rows_buf, acc.at[dstbuf.at[j]], add=True)
    return 0

  lax.fori_loop(0, NCHUNK, chunk, 0)
  plsc.subcore_barrier()
  _writeout(acc, out_hbm, cid, sid)


BN = 2048


def _combine_body(act, h_ref, p_ref, dg_ref, ws_ref, wn_ref, b_ref, o_ref):
  deg = jnp.maximum(dg_ref[0] + dg_ref[1], 1.0)
  neigh = (p_ref[0] + p_ref[1]) / deg
  out = (jnp.dot(h_ref[...], ws_ref[...], preferred_element_type=jnp.float32)
         + jnp.dot(neigh, wn_ref[...], preferred_element_type=jnp.float32)
         + b_ref[...])
  if act:
    out = jnp.maximum(out, 0.0)
  o_ref[...] = out


def _combine(h, p, dg, ws, wn, b, act):
  grid = (NPAD // BN,)
  return pl.pallas_call(
      functools.partial(_combine_body, act),
      grid=grid,
      in_specs=[
          pl.BlockSpec((BN, D), lambda i: (i, 0)),
          pl.BlockSpec((NC, BN, D), lambda i: (0, i, 0)),
          pl.BlockSpec((NC, BN, D), lambda i: (0, i, 0)),
          pl.BlockSpec((D, D), lambda i: (0, 0)),
          pl.BlockSpec((D, D), lambda i: (0, 0)),
          pl.BlockSpec((1, D), lambda i: (0, 0)),
      ],
      out_specs=pl.BlockSpec((BN, D), lambda i: (i, 0)),
      out_shape=jax.ShapeDtypeStruct((NPAD, D), jnp.float32),
  )(h, p, dg, ws, wn, b)


def kernel(features, edge_index, Ws0, Wn0, b0, Ws1, Wn1, b1, Ws2, Wn2, b2):
  src = edge_index[0]
  dst = edge_index[1]
  npadrows = EPTP - EPT
  pad_src = jnp.broadcast_to((jnp.arange(npadrows, dtype=jnp.int32) * 89) % N,
                             (NW, npadrows))
  pad_dst = jnp.broadcast_to(
      N + (jnp.arange(npadrows, dtype=jnp.int32) % (NPAD - N)), (NW, npadrows))
  src3 = jnp.concatenate([src.reshape(NW, EPT), pad_src], axis=1)
  src3 = src3.reshape(NW, NCHUNK, CH)
  dst3 = jnp.concatenate([dst.reshape(NW, EPT), pad_dst], axis=1)
  dst3 = dst3.reshape(NW, NCHUNK, CH)

  h = jnp.zeros((NPAD, D), jnp.float32).at[:N].set(features)
  dg = _deg_kernel(dst3)

  layers = ((Ws0, Wn0, b0, True), (Ws1, Wn1, b1, True), (Ws2, Wn2, b2, False))
  for ws, wn, b, act in layers:
    p = _agg_kernel(h, src3, dst3)
    h = _combine(h, p, dg, ws, wn, b.reshape(1, D), act)
  return h[:N]


# speedup vs baseline: 11.0258x; 1.3653x over previous
import functools

import jax
import jax.numpy as jnp
from jax import lax
from jax.experimental import pallas as pl
from jax.experimental.pallas import tpu as pltpu
from jax.experimental.pallas import tpu_sc as plsc

N = 10000
E = 320000
D = 128
NPAD = 10240
NC = 2
NS = 16
NW = NC * NS
EPT = E // NW
CH = 128
NBUF = 2
GRP = 8
NCHUNK = 80
NCHS = NCHUNK + GRP
EPTP = NCHUNK * CH
RPT = NPAD // NS

_MESH = plsc.VectorSubcoreMesh(
    core_axis_name="c", subcore_axis_name="s", num_cores=NC, num_subcores=NS)


def _fill(buf, rows, val):
  v = jnp.full((16,), val, jnp.float32)

  def body(i, _):
    for k in range(D // 16):
      buf[i, pl.ds(k * 16, 16)] = v
    return 0

  lax.fori_loop(0, rows, body, 0)


def _zero_acc(acc, rows_buf, sid):
  _fill(rows_buf, CH, 0.0)
  for k in range(RPT // CH):
    pltpu.sync_copy(rows_buf, acc.at[pl.ds(sid * RPT + k * CH, CH), :])


def _writeout(acc, out_hbm, cid, sid):
  pltpu.sync_copy(acc.at[pl.ds(sid * RPT, RPT), :],
                  out_hbm.at[cid, pl.ds(sid * RPT, RPT), :])


@functools.partial(
    pl.kernel,
    out_type=jax.ShapeDtypeStruct((NC, NPAD, D), jnp.float32),
    mesh=_MESH,
    scratch_types=[
        pltpu.VMEM((NCHUNK, CH), jnp.int32),
        pltpu.VMEM((CH, D), jnp.float32),
        pltpu.MemorySpace.VMEM_SHARED((NPAD, D), jnp.float32),
    ],
)
def _deg_kernel(dst3_hbm, out_hbm, dstbuf, rows_buf, acc):
  cid = lax.axis_index("c")
  sid = lax.axis_index("s")
  wid = cid * NS + sid
  pltpu.sync_copy(dst3_hbm.at[wid], dstbuf)
  _zero_acc(acc, rows_buf, sid)
  plsc.subcore_barrier()
  _fill(rows_buf, CH, 1.0)

  def chunk(j, _):
    pltpu.sync_copy(rows_buf, acc.at[dstbuf.at[j]], add=True)
    return 0

  lax.fori_loop(0, NCHUNK, chunk, 0)
  plsc.subcore_barrier()
  _writeout(acc, out_hbm, cid, sid)


@functools.partial(
    pl.kernel,
    out_type=jax.ShapeDtypeStruct((NC, NPAD, D), jnp.float32),
    mesh=_MESH,
    scratch_types=[
        pltpu.VMEM((3, GRP, CH), jnp.int32),
        pltpu.VMEM((GRP, CH), jnp.int32),
        [pltpu.VMEM((CH, D), jnp.float32)] * NBUF,
        pltpu.MemorySpace.VMEM_SHARED((NPAD, D), jnp.float32),
        [pltpu.SemaphoreType.DMA] * NBUF,
    ],
)
def _agg_kernel(h_hbm, src3_hbm, dst3_hbm, out_hbm, sbi, dbi, ring, acc, sems):
  cid = lax.axis_index("c")
  sid = lax.axis_index("s")
  wid = cid * NS + sid
  _zero_acc(acc, ring[0], sid)
  pltpu.sync_copy(src3_hbm.at[wid, pl.ds(0, GRP)], sbi.at[0])
  plsc.subcore_barrier()

  for b in range(NBUF):
    pltpu.async_copy(h_hbm.at[sbi.at[0, b]], ring[b], sems[b])

  def group(g, _):
    gp = g % 3
    pltpu.sync_copy(src3_hbm.at[wid, pl.ds((g + 1) * GRP, GRP)],
                    sbi.at[(g + 1) % 3])
    pltpu.sync_copy(dst3_hbm.at[wid, pl.ds(g * GRP, GRP)], dbi)
    for b in range(GRP):
      s = b % NBUF
      pltpu.make_async_copy(h_hbm.at[sbi.at[gp, b]], ring[s], sems[s]).wait()
      pltpu.sync_copy(ring[s], acc.at[dbi.at[b]], add=True)
      if b + NBUF < GRP:
        pltpu.async_copy(h_hbm.at[sbi.at[gp, b + NBUF]], ring[s], sems[s])
      else:
        pltpu.async_copy(h_hbm.at[sbi.at[(g + 1) % 3, b + NBUF - GRP]],
                         ring[s], sems[s])
    return 0

  lax.fori_loop(0, NCHUNK // GRP, group, 0)
  for b in range(NBUF):
    pltpu.make_async_copy(h_hbm.at[sbi.at[0, b]], ring[b], sems[b]).wait()

  plsc.subcore_barrier()
  _writeout(acc, out_hbm, cid, sid)


BN = 2048


def _combine_body(act, h_ref, p_ref, dg_ref, ws_ref, wn_ref, b_ref, o_ref):
  deg = jnp.maximum(dg_ref[0] + dg_ref[1], 1.0)
  neigh = (p_ref[0] + p_ref[1]) / deg
  out = (jnp.dot(h_ref[...], ws_ref[...], preferred_element_type=jnp.float32)
         + jnp.dot(neigh, wn_ref[...], preferred_element_type=jnp.float32)
         + b_ref[...])
  if act:
    out = jnp.maximum(out, 0.0)
  o_ref[...] = out


def _combine(h, p, dg, ws, wn, b, act):
  grid = (NPAD // BN,)
  return pl.pallas_call(
      functools.partial(_combine_body, act),
      grid=grid,
      in_specs=[
          pl.BlockSpec((BN, D), lambda i: (i, 0)),
          pl.BlockSpec((NC, BN, D), lambda i: (0, i, 0)),
          pl.BlockSpec((NC, BN, D), lambda i: (0, i, 0)),
          pl.BlockSpec((D, D), lambda i: (0, 0)),
          pl.BlockSpec((D, D), lambda i: (0, 0)),
          pl.BlockSpec((1, D), lambda i: (0, 0)),
      ],
      out_specs=pl.BlockSpec((BN, D), lambda i: (i, 0)),
      out_shape=jax.ShapeDtypeStruct((NPAD, D), jnp.float32),
  )(h, p, dg, ws, wn, b)


def kernel(features, edge_index, Ws0, Wn0, b0, Ws1, Wn1, b1, Ws2, Wn2, b2):
  src = edge_index[0]
  dst = edge_index[1]
  nps = NCHS * CH - EPT
  npd = EPTP - EPT
  pad_src = jnp.broadcast_to((jnp.arange(nps, dtype=jnp.int32) * 89) % N,
                             (NW, nps))
  pad_dst = jnp.broadcast_to(
      N + (jnp.arange(npd, dtype=jnp.int32) % (NPAD - N)), (NW, npd))
  src3 = jnp.concatenate([src.reshape(NW, EPT), pad_src], axis=1)
  src3 = src3.reshape(NW, NCHS, CH)
  dst3 = jnp.concatenate([dst.reshape(NW, EPT), pad_dst], axis=1)
  dst3 = dst3.reshape(NW, NCHUNK, CH)

  h = jnp.zeros((NPAD, D), jnp.float32).at[:N].set(features)
  dg = _deg_kernel(dst3)

  layers = ((Ws0, Wn0, b0, True), (Ws1, Wn1, b1, True), (Ws2, Wn2, b2, False))
  for ws, wn, b, act in layers:
    p = _agg_kernel(h, src3, dst3)
    h = _combine(h, p, dg, ws, wn, b.reshape(1, D), act)
  return h[:N]


# speedup vs baseline: 11.0329x; 1.0006x over previous
import functools

import jax
import jax.numpy as jnp
from jax import lax
from jax.experimental import pallas as pl
from jax.experimental.pallas import tpu as pltpu
from jax.experimental.pallas import tpu_sc as plsc

N = 10000
E = 320000
D = 128
NPAD = 10240
NC = 2
NS = 16
NW = NC * NS
EPT = E // NW
CH = 128
NBUF = 2
GRP = 8
NCHUNK = 80
NCHS = NCHUNK + GRP
EPTP = NCHUNK * CH
RPT = NPAD // NS

_MESH = plsc.VectorSubcoreMesh(
    core_axis_name="c", subcore_axis_name="s", num_cores=NC, num_subcores=NS)


def _fill(buf, rows, val):
  v = jnp.full((16,), val, jnp.float32)

  def body(i, _):
    for k in range(D // 16):
      buf[i, pl.ds(k * 16, 16)] = v
    return 0

  lax.fori_loop(0, rows, body, 0)


def _zero_acc(acc, rows_buf, sid):
  _fill(rows_buf, CH, 0.0)
  for k in range(RPT // CH):
    pltpu.sync_copy(rows_buf, acc.at[pl.ds(sid * RPT + k * CH, CH), :])


def _writeout(acc, out_hbm, cid, sid):
  pltpu.sync_copy(acc.at[pl.ds(sid * RPT, RPT), :],
                  out_hbm.at[cid, pl.ds(sid * RPT, RPT), :])


@functools.partial(
    pl.kernel,
    out_type=jax.ShapeDtypeStruct((NC, NPAD, D), jnp.float32),
    mesh=_MESH,
    scratch_types=[
        pltpu.VMEM((NCHUNK, CH), jnp.int32),
        pltpu.VMEM((CH, D), jnp.float32),
        pltpu.MemorySpace.VMEM_SHARED((NPAD, D), jnp.float32),
        pltpu.SemaphoreType.DMA,
    ],
)
def _deg_kernel(dst3_hbm, out_hbm, dstbuf, rows_buf, acc, sem):
  cid = lax.axis_index("c")
  sid = lax.axis_index("s")
  wid = cid * NS + sid
  pltpu.sync_copy(dst3_hbm.at[wid], dstbuf)
  _zero_acc(acc, rows_buf, sid)
  plsc.subcore_barrier()
  _fill(rows_buf, CH, 1.0)

  def group(g, _):
    jo = g * GRP
    for b in range(GRP):
      pltpu.async_copy(rows_buf, acc.at[dstbuf.at[jo + b]], sem, add=True)
    for b in range(GRP):
      pltpu.make_async_copy(rows_buf, acc.at[dstbuf.at[jo + b]], sem).wait()
    return 0

  lax.fori_loop(0, NCHUNK // GRP, group, 0)
  plsc.subcore_barrier()
  _writeout(acc, out_hbm, cid, sid)


@functools.partial(
    pl.kernel,
    out_type=jax.ShapeDtypeStruct((NC, NPAD, D), jnp.float32),
    mesh=_MESH,
    scratch_types=[
        pltpu.VMEM((3, GRP, CH), jnp.int32),
        pltpu.VMEM((GRP, CH), jnp.int32),
        [pltpu.VMEM((CH, D), jnp.float32)] * NBUF,
        pltpu.MemorySpace.VMEM_SHARED((NPAD, D), jnp.float32),
        [pltpu.SemaphoreType.DMA] * NBUF,
    ],
)
def _agg_kernel(h_hbm, src3_hbm, dst3_hbm, out_hbm, sbi, dbi, ring, acc, sems):
  cid = lax.axis_index("c")
  sid = lax.axis_index("s")
  wid = cid * NS + sid
  _zero_acc(acc, ring[0], sid)
  pltpu.sync_copy(src3_hbm.at[wid, pl.ds(0, GRP)], sbi.at[0])
  plsc.subcore_barrier()

  for b in range(NBUF):
    pltpu.async_copy(h_hbm.at[sbi.at[0, b]], ring[b], sems[b])

  def group(g, _):
    gp = g % 3
    pltpu.sync_copy(src3_hbm.at[wid, pl.ds((g + 1) * GRP, GRP)],
                    sbi.at[(g + 1) % 3])
    pltpu.sync_copy(dst3_hbm.at[wid, pl.ds(g * GRP, GRP)], dbi)
    for b in range(GRP):
      s = b % NBUF
      pltpu.make_async_copy(h_hbm.at[sbi.at[gp, b]], ring[s], sems[s]).wait()
      pltpu.sync_copy(ring[s], acc.at[dbi.at[b]], add=True)
      if b + NBUF < GRP:
        pltpu.async_copy(h_hbm.at[sbi.at[gp, b + NBUF]], ring[s], sems[s])
      else:
        pltpu.async_copy(h_hbm.at[sbi.at[(g + 1) % 3, b + NBUF - GRP]],
                         ring[s], sems[s])
    return 0

  lax.fori_loop(0, NCHUNK // GRP, group, 0)
  for b in range(NBUF):
    pltpu.make_async_copy(h_hbm.at[sbi.at[0, b]], ring[b], sems[b]).wait()

  plsc.subcore_barrier()
  _writeout(acc, out_hbm, cid, sid)


BN = 2048


def _combine_body(act, h_ref, p_ref, dg_ref, ws_ref, wn_ref, b_ref, o_ref):
  deg = jnp.maximum(dg_ref[0] + dg_ref[1], 1.0)
  neigh = (p_ref[0] + p_ref[1]) / deg
  out = (jnp.dot(h_ref[...], ws_ref[...], preferred_element_type=jnp.float32)
         + jnp.dot(neigh, wn_ref[...], preferred_element_type=jnp.float32)
         + b_ref[...])
  if act:
    out = jnp.maximum(out, 0.0)
  o_ref[...] = out


def _combine(h, p, dg, ws, wn, b, act):
  grid = (NPAD // BN,)
  return pl.pallas_call(
      functools.partial(_combine_body, act),
      grid=grid,
      in_specs=[
          pl.BlockSpec((BN, D), lambda i: (i, 0)),
          pl.BlockSpec((NC, BN, D), lambda i: (0, i, 0)),
          pl.BlockSpec((NC, BN, D), lambda i: (0, i, 0)),
          pl.BlockSpec((D, D), lambda i: (0, 0)),
          pl.BlockSpec((D, D), lambda i: (0, 0)),
          pl.BlockSpec((1, D), lambda i: (0, 0)),
      ],
      out_specs=pl.BlockSpec((BN, D), lambda i: (i, 0)),
      out_shape=jax.ShapeDtypeStruct((NPAD, D), jnp.float32),
  )(h, p, dg, ws, wn, b)


def kernel(features, edge_index, Ws0, Wn0, b0, Ws1, Wn1, b1, Ws2, Wn2, b2):
  src = edge_index[0]
  dst = edge_index[1]
  nps = NCHS * CH - EPT
  npd = EPTP - EPT
  pad_src = jnp.broadcast_to((jnp.arange(nps, dtype=jnp.int32) * 89) % N,
                             (NW, nps))
  pad_dst = jnp.broadcast_to(
      N + (jnp.arange(npd, dtype=jnp.int32) % (NPAD - N)), (NW, npd))
  src3 = jnp.concatenate([src.reshape(NW, EPT), pad_src], axis=1)
  src3 = src3.reshape(NW, NCHS, CH)
  dst3 = jnp.concatenate([dst.reshape(NW, EPT), pad_dst], axis=1)
  dst3 = dst3.reshape(NW, NCHUNK, CH)

  h = jnp.zeros((NPAD, D), jnp.float32).at[:N].set(features)
  dg = _deg_kernel(dst3)

  layers = ((Ws0, Wn0, b0, True), (Ws1, Wn1, b1, True), (Ws2, Wn2, b2, False))
  for ws, wn, b, act in layers:
    p = _agg_kernel(h, src3, dst3)
    h = _combine(h, p, dg, ws, wn, b.reshape(1, D), act)
  return h[:N]
